# baseline (speedup 1.0000x reference)
import functools

import jax
import jax.numpy as jnp
from jax import lax
from jax.experimental import pallas as pl
from jax.experimental.pallas import tpu as pltpu
from jax.experimental.pallas import tpu_sc as plsc

N = 10000
E = 320000
R = 32
NB = 8
D = 128

NC = 2
NS = 16
L = 16

CH = 80
EPT = E // (NC * NS)
HPT = E // NS
NR = N * R

_mesh = plsc.VectorSubcoreMesh(core_axis_name="c", subcore_axis_name="s")

_GD = lax.GatherDimensionNumbers(
    offset_dims=(), collapsed_slice_dims=(0,), start_index_map=(0,))


def _bcast_lane(vec, i):
    idx = jnp.full((L, 1), i, jnp.int32)
    return lax.gather(vec, idx, _GD, (1,),
                      mode=lax.GatherScatterMode.PROMISE_IN_BOUNDS)


_HRING = 5
_GRING = 5
NCH = EPT // CH

MWORDS = 3 * CH

@functools.partial(
    pl.kernel,
    out_type=jax.ShapeDtypeStruct((E * 3,), jnp.int32),
    mesh=_mesh,
    scratch_types=[
        pltpu.VMEM_SHARED((NR,), jnp.int32),
        pltpu.VMEM((2000,), jnp.int32),
        pltpu.VMEM((HPT,), jnp.int32),
        pltpu.VMEM((HPT,), jnp.int32),
        pltpu.VMEM((EPT,), jnp.int32),
        pltpu.VMEM((EPT,), jnp.int32),
        pltpu.VMEM((NCH * MWORDS,), jnp.int32),
        pltpu.VMEM((_HRING, CH), jnp.int32),
        pltpu.VMEM((CH,), jnp.int32),
        pltpu.SemaphoreType.DMA((_HRING,)),
        pltpu.SemaphoreType.DMA((_GRING,)),
    ],
)
def _sc_prep(src_hbm, dst_hbm, et_hbm, meta_hbm,
             cnt_sh, zb, dball, tball, sball, segE, mb,
             segring, oneb, hsems, gsems):
    cid = lax.axis_index("c")
    sid = lax.axis_index("s")

    def z16(i, _):
        zb[pl.ds(i * L, L)] = jnp.zeros((L,), jnp.int32)
        return 0
    lax.fori_loop(0, 2000 // L, z16, 0)

    for i in range(HPT // 2000):
        pltpu.async_copy(zb, cnt_sh.at[pl.ds(sid * HPT + i * 2000, 2000)],
                         gsems.at[0])
    for i in range(HPT // 2000):
        pltpu.make_async_copy(zb, cnt_sh.at[pl.ds(sid * HPT, 2000)],
                              gsems.at[0]).wait()

    def o16(i, _):
        oneb[pl.ds(i * L, L)] = jnp.ones((L,), jnp.int32)
        return 0
    lax.fori_loop(0, CH // L, o16, 0)

    plsc.subcore_barrier()

    hbase = sid * HPT
    pltpu.sync_copy(dst_hbm.at[pl.ds(hbase, HPT)], dball)
    pltpu.sync_copy(et_hbm.at[pl.ds(hbase, HPT)], tball)

    nhc = HPT // CH
    def hist_outer(st, _):
        for b in range(_HRING):
            k = st * _HRING + b
            @pl.when(k >= _HRING)
            def _():
                pltpu.make_async_copy(
                    oneb, cnt_sh.at[segring.at[b]], hsems.at[b]).wait()
            off = k * CH
            for j in range(CH // L):
                sl = pl.ds(j * L, L)
                segring[b, sl] = dball[pl.ds(off + j * L, L)] * R + tball[pl.ds(off + j * L, L)]
            pltpu.async_copy(oneb, cnt_sh.at[segring.at[b]], hsems.at[b], add=True)
        return 0
    lax.fori_loop(0, nhc // _HRING, hist_outer, 0)
    for b in range(_HRING):
        pltpu.make_async_copy(oneb, cnt_sh.at[segring.at[b]], hsems.at[b]).wait()

    plsc.subcore_barrier()

    wid = cid * NS + sid
    ebase = wid * EPT
    pltpu.sync_copy(dst_hbm.at[pl.ds(ebase, EPT)], dball.at[pl.ds(0, EPT)])
    pltpu.sync_copy(et_hbm.at[pl.ds(ebase, EPT)], tball.at[pl.ds(0, EPT)])
    pltpu.sync_copy(src_hbm.at[pl.ds(ebase, EPT)], sball)

    def segs(i, _):
        sl = pl.ds(i * L, L)
        segE[sl] = dball[sl] * R + tball[sl]
        return 0
    lax.fori_loop(0, EPT // L, segs, 0)

    def g_outer(st, _):
        for b in range(_GRING):
            k = st * _GRING + b
            @pl.when(k >= _GRING)
            def _():
                pltpu.make_async_copy(
                    cnt_sh.at[segE.at[pl.ds(0, CH)]],
                    mb.at[pl.ds(2 * CH, CH)], gsems.at[b]).wait()
            off = k * CH
            pltpu.async_copy(cnt_sh.at[segE.at[pl.ds(off, CH)]],
                             mb.at[pl.ds(k * MWORDS + 2 * CH, CH)], gsems.at[b])
        return 0
    lax.fori_loop(0, NCH // _GRING, g_outer, 0)

    def meta_i(i, _):
        for j in range(CH // L):
            esl = pl.ds(i * CH + j * L, L)
            mb[pl.ds(i * MWORDS + j * L, L)] = tball[esl] * N + sball[esl]
            mb[pl.ds(i * MWORDS + CH + j * L, L)] = dball[esl]
        return 0
    lax.fori_loop(0, NCH, meta_i, 0)

    for b in range(_GRING):
        pltpu.make_async_copy(cnt_sh.at[segE.at[pl.ds(0, CH)]],
                              mb.at[pl.ds(2 * CH, CH)], gsems.at[b]).wait()

    pltpu.sync_copy(mb, meta_hbm.at[pl.ds(wid * NCH * MWORDS, NCH * MWORDS)])


NPAD = 10240
_ZROWS = 32
_NBUF = 4

@functools.partial(
    pl.kernel,
    out_type=jax.ShapeDtypeStruct((NC, NPAD, D), jnp.float32),
    mesh=_mesh,
    scratch_types=[
        pltpu.VMEM_SHARED((NPAD, D), jnp.float32),
        pltpu.VMEM((_ZROWS, D), jnp.float32),
        pltpu.VMEM((_NBUF * MWORDS,), jnp.int32),
        pltpu.VMEM((_NBUF, CH), jnp.int32),
        pltpu.VMEM((_NBUF, CH, D), jnp.float32),
        pltpu.SemaphoreType.DMA((_NBUF,)),
        pltpu.SemaphoreType.DMA((_NBUF,)),
        pltpu.SemaphoreType.DMA((_NBUF,)),
        pltpu.SemaphoreType.DMA,
    ],
)
def _sc_edge(y_hbm, meta_hbm, p_hbm,
             acc, zrows, mring, dring, rbufs, msems, gsems, ssems, wsem):
    cid = lax.axis_index("c")
    sid = lax.axis_index("s")

    def zrow(i, _):
        for j in range(D // L):
            zrows[i, pl.ds(j * L, L)] = jnp.zeros((L,), jnp.float32)
        return 0
    lax.fori_loop(0, _ZROWS, zrow, 0)

    npt = NPAD // NS
    for q in range(npt // _ZROWS):
        pltpu.async_copy(zrows, acc.at[pl.ds(sid * npt + q * _ZROWS, _ZROWS)],
                         wsem)
    for q in range(npt // _ZROWS):
        pltpu.make_async_copy(zrows, acc.at[pl.ds(sid * npt, _ZROWS)],
                              wsem).wait()

    plsc.subcore_barrier()

    t0 = (cid * NS + sid) * NCH * MWORDS

    def m_start(k, b):
        pltpu.async_copy(meta_hbm.at[pl.ds(t0 + k * MWORDS, MWORDS)],
                         mring.at[pl.ds(b * MWORDS, MWORDS)], msems.at[b])

    def m_wait(b):
        pltpu.make_async_copy(meta_hbm.at[pl.ds(0, MWORDS)],
                              mring.at[pl.ds(b * MWORDS, MWORDS)],
                              msems.at[b]).wait()

    def g_start(k, b):
        del k
        pltpu.async_copy(y_hbm.at[mring.at[pl.ds(b * MWORDS, CH)]],
                         rbufs.at[b], gsems.at[b])

    def g_wait(b):
        pltpu.make_async_copy(y_hbm.at[mring.at[pl.ds(b * MWORDS, CH)]],
                              rbufs.at[b], gsems.at[b]).wait()

    def s_wait(b):
        pltpu.make_async_copy(rbufs.at[b], acc.at[dring.at[b]],
                              ssems.at[b]).wait()

    m_start(0, 0)
    m_start(1, 1)
    m_start(2, 2)
    m_wait(0)
    g_start(0, 0)
    m_wait(1)
    g_start(1, 1)

    def iter_body(k, b):
        g_wait(b)

        def scale_group(g, _):
            sv16 = 1.0 / mring[pl.ds(b * MWORDS + 2 * CH + g * L, L)].astype(jnp.float32)
            for ii in range(L):
                svi = _bcast_lane(sv16, ii)
                row = g * L + ii
                for j in range(D // L):
                    sl = pl.ds(j * L, L)
                    rbufs[b, row, sl] = rbufs[b, row, sl] * svi
            return 0
        lax.fori_loop(0, CH // L, scale_group, 0)

        for j in range(CH // L):
            sl = pl.ds(j * L, L)
            dring[b, sl] = mring[pl.ds(b * MWORDS + CH + j * L, L)]
        pltpu.async_copy(rbufs.at[b], acc.at[dring.at[b]],
                         ssems.at[b], add=True)

        nmb = (b + 3) % _NBUF
        @pl.when(k + 3 < NCH)
        def _():
            m_start(k + 3, nmb)

        ngb = (b + 2) % _NBUF
        @pl.when(k >= 2)
        def _():
            s_wait(ngb)
        @pl.when(k + 2 < NCH)
        def _():
            m_wait(ngb)
            g_start(k + 2, ngb)

    def outer(st, _):
        for b in range(_NBUF):
            iter_body(st * _NBUF + b, b)
        return 0
    lax.fori_loop(0, (NCH - 1) // _NBUF, outer, 0)
    iter_body(jnp.int32(NCH - 1), (NCH - 1) % _NBUF)

    s_wait((NCH - 2) % _NBUF)
    s_wait((NCH - 1) % _NBUF)

    plsc.subcore_barrier()

    for q in range(npt // _ZROWS):
        r0 = sid * npt + q * _ZROWS
        pltpu.async_copy(acc.at[pl.ds(r0, _ZROWS)],
                         p_hbm.at[cid, pl.ds(r0, _ZROWS)], wsem)
    for q in range(npt // _ZROWS):
        pltpu.make_async_copy(acc.at[pl.ds(sid * npt, _ZROWS)],
                              p_hbm.at[cid, pl.ds(sid * npt, _ZROWS)],
                              wsem).wait()


def _y_body(comp_ref, basis_ref, x_ref, y_ref):
    r = pl.program_id(0)
    w = comp_ref[r, 0] * basis_ref[0]
    for b in range(1, NB):
        w = w + comp_ref[r, b] * basis_ref[b]
    y_ref[...] = jnp.dot(x_ref[...], w, preferred_element_type=jnp.float32)


def _y_call(x, comp, basis):
    return pl.pallas_call(
        _y_body,
        grid=(R,),
        in_specs=[
            pl.BlockSpec(memory_space=pltpu.SMEM),
            pl.BlockSpec((NB, D, D), lambda r: (0, 0, 0)),
            pl.BlockSpec((N, D), lambda r: (0, 0)),
        ],
        out_specs=pl.BlockSpec((N, D), lambda r: (r, 0)),
        out_shape=jax.ShapeDtypeStruct((R * N, D), jnp.float32),
    )(comp, basis, x)


_BN = 1000

def _epi_body(relu, p_ref, x_ref, root_ref, bias_ref, o_ref):
    acc = (p_ref[0] + p_ref[1] + bias_ref[0]
           + jnp.dot(x_ref[...], root_ref[...], preferred_element_type=jnp.float32))
    if relu:
        acc = jnp.maximum(acc, 0.0)
    o_ref[...] = acc


def _epi_call(p, x, root, bias, relu):
    return pl.pallas_call(
        functools.partial(_epi_body, relu),
        grid=(N // _BN,),
        in_specs=[
            pl.BlockSpec((NC, _BN, D), lambda i: (0, i, 0)),
            pl.BlockSpec((_BN, D), lambda i: (i, 0)),
            pl.BlockSpec((D, D), lambda i: (0, 0)),
            pl.BlockSpec((1, D), lambda i: (0, 0)),
        ],
        out_specs=pl.BlockSpec((_BN, D), lambda i: (i, 0)),
        out_shape=jax.ShapeDtypeStruct((N, D), jnp.float32),
    )(p, x, root, bias.reshape(1, D))


def kernel(edge_index, edge_type, embedding, basis1, comp1, root1, bias1,
           basis2, comp2, root2, bias2):
    src = edge_index[0]
    dst = edge_index[1]
    meta = _sc_prep(src, dst, edge_type)

    y1 = _y_call(embedding, comp1, basis1)
    p1 = _sc_edge(y1, meta)
    x2 = _epi_call(p1, embedding, root1, bias1, relu=True)

    y2 = _y_call(x2, comp2, basis2)
    p2 = _sc_edge(y2, meta)
    return _epi_call(p2, x2, root2, bias2, relu=False)


# speedup vs baseline: 1.0013x; 1.0013x over previous
import functools

import jax
import jax.numpy as jnp
from jax import lax
from jax.experimental import pallas as pl
from jax.experimental.pallas import tpu as pltpu
from jax.experimental.pallas import tpu_sc as plsc

N = 10000
E = 320000
R = 32
NB = 8
D = 128

NC = 2
NS = 16
L = 16

CH = 80
EPT = E // (NC * NS)
HPT = E // NS
NR = N * R

_mesh = plsc.VectorSubcoreMesh(core_axis_name="c", subcore_axis_name="s")

_GD = lax.GatherDimensionNumbers(
    offset_dims=(), collapsed_slice_dims=(0,), start_index_map=(0,))


def _bcast_lane(vec, i):
    idx = jnp.full((L, 1), i, jnp.int32)
    return lax.gather(vec, idx, _GD, (1,),
                      mode=lax.GatherScatterMode.PROMISE_IN_BOUNDS)


_HRING = 5
_GRING = 5
NCH = EPT // CH

MWORDS = 3 * CH

@functools.partial(
    pl.kernel,
    out_type=jax.ShapeDtypeStruct((E * 3,), jnp.int32),
    mesh=_mesh,
    scratch_types=[
        pltpu.VMEM_SHARED((NR,), jnp.int32),
        pltpu.VMEM((2000,), jnp.int32),
        pltpu.VMEM((HPT,), jnp.int32),
        pltpu.VMEM((HPT,), jnp.int32),
        pltpu.VMEM((EPT,), jnp.int32),
        pltpu.VMEM((EPT,), jnp.int32),
        pltpu.VMEM((NCH * MWORDS,), jnp.int32),
        pltpu.VMEM((_HRING, CH), jnp.int32),
        pltpu.VMEM((CH,), jnp.int32),
        pltpu.SemaphoreType.DMA((_HRING,)),
        pltpu.SemaphoreType.DMA((_GRING,)),
    ],
)
def _sc_prep(src_hbm, dst_hbm, et_hbm, meta_hbm,
             cnt_sh, zb, dball, tball, sball, segE, mb,
             segring, oneb, hsems, gsems):
    cid = lax.axis_index("c")
    sid = lax.axis_index("s")

    def z16(i, _):
        zb[pl.ds(i * L, L)] = jnp.zeros((L,), jnp.int32)
        return 0
    lax.fori_loop(0, 2000 // L, z16, 0)

    for i in range(HPT // 2000):
        pltpu.async_copy(zb, cnt_sh.at[pl.ds(sid * HPT + i * 2000, 2000)],
                         gsems.at[0])
    for i in range(HPT // 2000):
        pltpu.make_async_copy(zb, cnt_sh.at[pl.ds(sid * HPT, 2000)],
                              gsems.at[0]).wait()

    def o16(i, _):
        oneb[pl.ds(i * L, L)] = jnp.ones((L,), jnp.int32)
        return 0
    lax.fori_loop(0, CH // L, o16, 0)

    plsc.subcore_barrier()

    hbase = sid * HPT
    pltpu.sync_copy(dst_hbm.at[pl.ds(hbase, HPT)], dball)
    pltpu.sync_copy(et_hbm.at[pl.ds(hbase, HPT)], tball)

    nhc = HPT // CH
    def hist_outer(st, _):
        for b in range(_HRING):
            k = st * _HRING + b
            @pl.when(k >= _HRING)
            def _():
                pltpu.make_async_copy(
                    oneb, cnt_sh.at[segring.at[b]], hsems.at[b]).wait()
            off = k * CH
            for j in range(CH // L):
                sl = pl.ds(j * L, L)
                segring[b, sl] = dball[pl.ds(off + j * L, L)] * R + tball[pl.ds(off + j * L, L)]
            pltpu.async_copy(oneb, cnt_sh.at[segring.at[b]], hsems.at[b], add=True)
        return 0
    lax.fori_loop(0, nhc // _HRING, hist_outer, 0)
    for b in range(_HRING):
        pltpu.make_async_copy(oneb, cnt_sh.at[segring.at[b]], hsems.at[b]).wait()

    plsc.subcore_barrier()

    wid = cid * NS + sid
    ebase = wid * EPT
    pltpu.sync_copy(dst_hbm.at[pl.ds(ebase, EPT)], dball.at[pl.ds(0, EPT)])
    pltpu.sync_copy(et_hbm.at[pl.ds(ebase, EPT)], tball.at[pl.ds(0, EPT)])
    pltpu.sync_copy(src_hbm.at[pl.ds(ebase, EPT)], sball)

    def segs(i, _):
        sl = pl.ds(i * L, L)
        segE[sl] = dball[sl] * R + tball[sl]
        return 0
    lax.fori_loop(0, EPT // L, segs, 0)

    def g_outer(st, _):
        for b in range(_GRING):
            k = st * _GRING + b
            @pl.when(k >= _GRING)
            def _():
                pltpu.make_async_copy(
                    cnt_sh.at[segE.at[pl.ds(0, CH)]],
                    mb.at[pl.ds(2 * CH, CH)], gsems.at[b]).wait()
            off = k * CH
            pltpu.async_copy(cnt_sh.at[segE.at[pl.ds(off, CH)]],
                             mb.at[pl.ds(k * MWORDS + 2 * CH, CH)], gsems.at[b])
        return 0
    lax.fori_loop(0, NCH // _GRING, g_outer, 0)

    def meta_i(i, _):
        for j in range(CH // L):
            esl = pl.ds(i * CH + j * L, L)
            mb[pl.ds(i * MWORDS + j * L, L)] = tball[esl] * N + sball[esl]
            mb[pl.ds(i * MWORDS + CH + j * L, L)] = dball[esl]
        return 0
    lax.fori_loop(0, NCH, meta_i, 0)

    for b in range(_GRING):
        pltpu.make_async_copy(cnt_sh.at[segE.at[pl.ds(0, CH)]],
                              mb.at[pl.ds(2 * CH, CH)], gsems.at[b]).wait()

    pltpu.sync_copy(mb, meta_hbm.at[pl.ds(wid * NCH * MWORDS, NCH * MWORDS)])


NPAD = 10240
_ZROWS = 32
_NBUF = 4

@functools.partial(
    pl.kernel,
    out_type=jax.ShapeDtypeStruct((NC, NPAD, D), jnp.float32),
    mesh=_mesh,
    scratch_types=[
        pltpu.VMEM_SHARED((NPAD, D), jnp.float32),
        pltpu.VMEM((_ZROWS, D), jnp.float32),
        pltpu.VMEM((_NBUF * MWORDS,), jnp.int32),
        pltpu.VMEM((_NBUF, CH), jnp.int32),
        pltpu.VMEM((_NBUF, CH, D), jnp.float32),
        pltpu.SemaphoreType.DMA((_NBUF,)),
        pltpu.SemaphoreType.DMA((_NBUF,)),
        pltpu.SemaphoreType.DMA((_NBUF,)),
        pltpu.SemaphoreType.DMA,
    ],
)
def _sc_edge(y_hbm, meta_hbm, p_hbm,
             acc, zrows, mring, dring, rbufs, msems, gsems, ssems, wsem):
    cid = lax.axis_index("c")
    sid = lax.axis_index("s")

    def zrow(i, _):
        for j in range(D // L):
            zrows[i, pl.ds(j * L, L)] = jnp.zeros((L,), jnp.float32)
        return 0
    lax.fori_loop(0, _ZROWS, zrow, 0)

    npt = NPAD // NS
    for q in range(npt // _ZROWS):
        pltpu.async_copy(zrows, acc.at[pl.ds(sid * npt + q * _ZROWS, _ZROWS)],
                         wsem)
    for q in range(npt // _ZROWS):
        pltpu.make_async_copy(zrows, acc.at[pl.ds(sid * npt, _ZROWS)],
                              wsem).wait()

    plsc.subcore_barrier()

    t0 = (cid * NS + sid) * NCH * MWORDS

    def m_start(k, b):
        pltpu.async_copy(meta_hbm.at[pl.ds(t0 + k * MWORDS, MWORDS)],
                         mring.at[pl.ds(b * MWORDS, MWORDS)], msems.at[b])

    def m_wait(b):
        pltpu.make_async_copy(meta_hbm.at[pl.ds(0, MWORDS)],
                              mring.at[pl.ds(b * MWORDS, MWORDS)],
                              msems.at[b]).wait()

    def g_start(k, b):
        del k
        pltpu.async_copy(y_hbm.at[mring.at[pl.ds(b * MWORDS, CH)]],
                         rbufs.at[b], gsems.at[b])

    def g_wait(b):
        pltpu.make_async_copy(y_hbm.at[mring.at[pl.ds(b * MWORDS, CH)]],
                              rbufs.at[b], gsems.at[b]).wait()

    def s_wait(b):
        pltpu.make_async_copy(rbufs.at[b], acc.at[dring.at[b]],
                              ssems.at[b]).wait()

    m_start(0, 0)
    m_start(1, 1)
    m_start(2, 2)
    m_wait(0)
    g_start(0, 0)
    m_wait(1)
    g_start(1, 1)

    def iter_body(k, b):
        g_wait(b)

        def scale_group(g, _):
            sv16 = 1.0 / mring[pl.ds(b * MWORDS + 2 * CH + g * L, L)].astype(jnp.float32)
            for ii in range(L):
                svi = _bcast_lane(sv16, ii)
                row = g * L + ii
                for j in range(D // L):
                    sl = pl.ds(j * L, L)
                    rbufs[b, row, sl] = rbufs[b, row, sl] * svi
            return 0
        lax.fori_loop(0, CH // L, scale_group, 0)

        for j in range(CH // L):
            sl = pl.ds(j * L, L)
            dring[b, sl] = mring[pl.ds(b * MWORDS + CH + j * L, L)]
        pltpu.async_copy(rbufs.at[b], acc.at[dring.at[b]],
                         ssems.at[b], add=True)

        nmb = (b + 3) % _NBUF
        @pl.when(k + 3 < NCH)
        def _():
            m_start(k + 3, nmb)

        ngb = (b + 2) % _NBUF
        @pl.when(k >= 2)
        def _():
            s_wait(ngb)
        @pl.when(k + 2 < NCH)
        def _():
            m_wait(ngb)
            g_start(k + 2, ngb)

    def outer(st, _):
        for b in range(_NBUF):
            iter_body(st * _NBUF + b, b)
        return 0
    lax.fori_loop(0, (NCH - 1) // _NBUF, outer, 0)
    iter_body(jnp.int32(NCH - 1), (NCH - 1) % _NBUF)

    s_wait((NCH - 2) % _NBUF)
    s_wait((NCH - 1) % _NBUF)

    plsc.subcore_barrier()

    for q in range(npt // _ZROWS):
        r0 = sid * npt + q * _ZROWS
        pltpu.async_copy(acc.at[pl.ds(r0, _ZROWS)],
                         p_hbm.at[cid, pl.ds(r0, _ZROWS)], wsem)
    for q in range(npt // _ZROWS):
        pltpu.make_async_copy(acc.at[pl.ds(sid * npt, _ZROWS)],
                              p_hbm.at[cid, pl.ds(sid * npt, _ZROWS)],
                              wsem).wait()


def _y_body(comp_ref, basis_ref, x_ref, y_ref):
    r = pl.program_id(0)
    w = comp_ref[r, 0] * basis_ref[0]
    for b in range(1, NB):
        w = w + comp_ref[r, b] * basis_ref[b]
    y_ref[...] = jnp.dot(x_ref[...], w, preferred_element_type=jnp.float32)


def _y_call(x, comp, basis):
    return pl.pallas_call(
        _y_body,
        grid=(R,),
        in_specs=[
            pl.BlockSpec(memory_space=pltpu.SMEM),
            pl.BlockSpec((NB, D, D), lambda r: (0, 0, 0)),
            pl.BlockSpec((N, D), lambda r: (0, 0)),
        ],
        out_specs=pl.BlockSpec((N, D), lambda r: (r, 0)),
        out_shape=jax.ShapeDtypeStruct((R * N, D), jnp.float32),
    )(comp, basis, x)


_BN = 1000

def _epi_body(relu, p_ref, x_ref, root_ref, bias_ref, o_ref):
    acc = (p_ref[0] + p_ref[1] + bias_ref[0]
           + jnp.dot(x_ref[...], root_ref[...], preferred_element_type=jnp.float32))
    if relu:
        acc = jnp.maximum(acc, 0.0)
    o_ref[...] = acc


def _epi_call(p, x, root, bias, relu):
    return pl.pallas_call(
        functools.partial(_epi_body, relu),
        grid=(N // _BN,),
        in_specs=[
            pl.BlockSpec((NC, _BN, D), lambda i: (0, i, 0)),
            pl.BlockSpec((_BN, D), lambda i: (i, 0)),
            pl.BlockSpec((D, D), lambda i: (0, 0)),
            pl.BlockSpec((1, D), lambda i: (0, 0)),
        ],
        out_specs=pl.BlockSpec((_BN, D), lambda i: (i, 0)),
        out_shape=jax.ShapeDtypeStruct((N, D), jnp.float32),
    )(p, x, root, bias.reshape(1, D))


def kernel(edge_index, edge_type, embedding, basis1, comp1, root1, bias1,
           basis2, comp2, root2, bias2):
    src = edge_index[0]
    dst = edge_index[1]
    y1 = _y_call(embedding, comp1, basis1)
    meta = _sc_prep(src, dst, edge_type)

    p1 = _sc_edge(y1, meta)
    x2 = _epi_call(p1, embedding, root1, bias1, relu=True)

    y2 = _y_call(x2, comp2, basis2)
    p2 = _sc_edge(y2, meta)
    return _epi_call(p2, x2, root2, bias2, relu=False)


# speedup vs baseline: 1.0424x; 1.0411x over previous
import functools

import jax
import jax.numpy as jnp
from jax import lax
from jax.experimental import pallas as pl
from jax.experimental.pallas import tpu as pltpu
from jax.experimental.pallas import tpu_sc as plsc

N = 10000
E = 320000
R = 32
NB = 8
D = 128

NC = 2
NS = 16
L = 16

CH = 80
EPT = E // (NC * NS)
HPT = E // NS
NR = N * R

_mesh = plsc.VectorSubcoreMesh(core_axis_name="c", subcore_axis_name="s")

_GD = lax.GatherDimensionNumbers(
    offset_dims=(), collapsed_slice_dims=(0,), start_index_map=(0,))


def _bcast_lane(vec, i):
    idx = jnp.full((L, 1), i, jnp.int32)
    return lax.gather(vec, idx, _GD, (1,),
                      mode=lax.GatherScatterMode.PROMISE_IN_BOUNDS)


_HRING = 5
_GRING = 5
NCH = EPT // CH

MWORDS = 3 * CH

@functools.partial(
    pl.kernel,
    out_type=jax.ShapeDtypeStruct((E * 3,), jnp.int32),
    mesh=_mesh,
    scratch_types=[
        pltpu.VMEM_SHARED((NR,), jnp.int32),
        pltpu.VMEM((2000,), jnp.int32),
        pltpu.VMEM((HPT,), jnp.int32),
        pltpu.VMEM((HPT,), jnp.int32),
        pltpu.VMEM((EPT,), jnp.int32),
        pltpu.VMEM((EPT,), jnp.int32),
        pltpu.VMEM((NCH * MWORDS,), jnp.int32),
        pltpu.VMEM((_HRING, CH), jnp.int32),
        pltpu.VMEM((CH,), jnp.int32),
        pltpu.SemaphoreType.DMA((_HRING,)),
        pltpu.SemaphoreType.DMA((_GRING,)),
    ],
)
def _sc_prep(src_hbm, dst_hbm, et_hbm, meta_hbm,
             cnt_sh, zb, dball, tball, sball, segE, mb,
             segring, oneb, hsems, gsems):
    cid = lax.axis_index("c")
    sid = lax.axis_index("s")

    def z16(i, _):
        zb[pl.ds(i * L, L)] = jnp.zeros((L,), jnp.int32)
        return 0
    lax.fori_loop(0, 2000 // L, z16, 0)

    for i in range(HPT // 2000):
        pltpu.async_copy(zb, cnt_sh.at[pl.ds(sid * HPT + i * 2000, 2000)],
                         gsems.at[0])
    for i in range(HPT // 2000):
        pltpu.make_async_copy(zb, cnt_sh.at[pl.ds(sid * HPT, 2000)],
                              gsems.at[0]).wait()

    def o16(i, _):
        oneb[pl.ds(i * L, L)] = jnp.ones((L,), jnp.int32)
        return 0
    lax.fori_loop(0, CH // L, o16, 0)

    plsc.subcore_barrier()

    hbase = sid * HPT
    pltpu.sync_copy(dst_hbm.at[pl.ds(hbase, HPT)], dball)
    pltpu.sync_copy(et_hbm.at[pl.ds(hbase, HPT)], tball)

    nhc = HPT // CH
    def hist_outer(st, _):
        for b in range(_HRING):
            k = st * _HRING + b
            @pl.when(k >= _HRING)
            def _():
                pltpu.make_async_copy(
                    oneb, cnt_sh.at[segring.at[b]], hsems.at[b]).wait()
            off = k * CH
            for j in range(CH // L):
                sl = pl.ds(j * L, L)
                segring[b, sl] = dball[pl.ds(off + j * L, L)] * R + tball[pl.ds(off + j * L, L)]
            pltpu.async_copy(oneb, cnt_sh.at[segring.at[b]], hsems.at[b], add=True)
        return 0
    lax.fori_loop(0, nhc // _HRING, hist_outer, 0)
    for b in range(_HRING):
        pltpu.make_async_copy(oneb, cnt_sh.at[segring.at[b]], hsems.at[b]).wait()

    plsc.subcore_barrier()

    wid = cid * NS + sid
    ebase = wid * EPT
    pltpu.sync_copy(dst_hbm.at[pl.ds(ebase, EPT)], dball.at[pl.ds(0, EPT)])
    pltpu.sync_copy(et_hbm.at[pl.ds(ebase, EPT)], tball.at[pl.ds(0, EPT)])
    pltpu.sync_copy(src_hbm.at[pl.ds(ebase, EPT)], sball)

    def segs(i, _):
        sl = pl.ds(i * L, L)
        segE[sl] = dball[sl] * R + tball[sl]
        return 0
    lax.fori_loop(0, EPT // L, segs, 0)

    def g_outer(st, _):
        for b in range(_GRING):
            k = st * _GRING + b
            @pl.when(k >= _GRING)
            def _():
                pltpu.make_async_copy(
                    cnt_sh.at[segE.at[pl.ds(0, CH)]],
                    mb.at[pl.ds(2 * CH, CH)], gsems.at[b]).wait()
            off = k * CH
            pltpu.async_copy(cnt_sh.at[segE.at[pl.ds(off, CH)]],
                             mb.at[pl.ds(k * MWORDS + 2 * CH, CH)], gsems.at[b])
        return 0
    lax.fori_loop(0, NCH // _GRING, g_outer, 0)

    def meta_i(i, _):
        for j in range(CH // L):
            esl = pl.ds(i * CH + j * L, L)
            mb[pl.ds(i * MWORDS + j * L, L)] = tball[esl] * N + sball[esl]
            mb[pl.ds(i * MWORDS + CH + j * L, L)] = dball[esl]
        return 0
    lax.fori_loop(0, NCH, meta_i, 0)

    for b in range(_GRING):
        pltpu.make_async_copy(cnt_sh.at[segE.at[pl.ds(0, CH)]],
                              mb.at[pl.ds(2 * CH, CH)], gsems.at[b]).wait()

    pltpu.sync_copy(mb, meta_hbm.at[pl.ds(wid * NCH * MWORDS, NCH * MWORDS)])


NPAD = 10240
_ZROWS = 32
_NBUF = 4

@functools.partial(
    pl.kernel,
    out_type=jax.ShapeDtypeStruct((NC, NPAD, D), jnp.float32),
    mesh=_mesh,
    scratch_types=[
        pltpu.VMEM_SHARED((NPAD, D), jnp.float32),
        pltpu.VMEM((_ZROWS, D), jnp.float32),
        pltpu.VMEM((_NBUF * MWORDS,), jnp.int32),
        pltpu.VMEM((_NBUF, CH), jnp.int32),
        pltpu.VMEM((_NBUF, CH, D), jnp.float32),
        pltpu.SemaphoreType.DMA((_NBUF,)),
        pltpu.SemaphoreType.DMA((_NBUF,)),
        pltpu.SemaphoreType.DMA((_NBUF,)),
        pltpu.SemaphoreType.DMA,
    ],
)
def _sc_edge(y_hbm, meta_hbm, p_hbm,
             acc, zrows, mring, dring, rbufs, msems, gsems, ssems, wsem):
    cid = lax.axis_index("c")
    sid = lax.axis_index("s")

    def zrow(i, _):
        for j in range(D // L):
            zrows[i, pl.ds(j * L, L)] = jnp.zeros((L,), jnp.float32)
        return 0
    lax.fori_loop(0, _ZROWS, zrow, 0)

    npt = NPAD // NS
    for q in range(npt // _ZROWS):
        pltpu.async_copy(zrows, acc.at[pl.ds(sid * npt + q * _ZROWS, _ZROWS)],
                         wsem)
    for q in range(npt // _ZROWS):
        pltpu.make_async_copy(zrows, acc.at[pl.ds(sid * npt, _ZROWS)],
                              wsem).wait()

    plsc.subcore_barrier()

    t0 = (cid * NS + sid) * NCH * MWORDS

    def m_start(k, b):
        pltpu.async_copy(meta_hbm.at[pl.ds(t0 + k * MWORDS, MWORDS)],
                         mring.at[pl.ds(b * MWORDS, MWORDS)], msems.at[b])

    def m_wait(b):
        pltpu.make_async_copy(meta_hbm.at[pl.ds(0, MWORDS)],
                              mring.at[pl.ds(b * MWORDS, MWORDS)],
                              msems.at[b]).wait()

    def g_start(k, b):
        del k
        pltpu.async_copy(y_hbm.at[mring.at[pl.ds(b * MWORDS, CH)]],
                         rbufs.at[b], gsems.at[b])

    def g_wait(b):
        pltpu.make_async_copy(y_hbm.at[mring.at[pl.ds(b * MWORDS, CH)]],
                              rbufs.at[b], gsems.at[b]).wait()

    def s_wait(b):
        pltpu.make_async_copy(rbufs.at[b], acc.at[dring.at[b]],
                              ssems.at[b]).wait()

    m_start(0, 0)
    m_start(1, 1)
    m_start(2, 2)
    m_wait(0)
    g_start(0, 0)
    m_wait(1)
    g_start(1, 1)

    def iter_body(k, b):
        g_wait(b)

        nmb = (b + 3) % _NBUF
        @pl.when(k + 3 < NCH)
        def _():
            m_start(k + 3, nmb)

        ngb = (b + 2) % _NBUF
        @pl.when(k >= 2)
        def _():
            s_wait(ngb)
        @pl.when(k + 2 < NCH)
        def _():
            m_wait(ngb)
            g_start(k + 2, ngb)

        def scale_group(g, _):
            sv16 = 1.0 / mring[pl.ds(b * MWORDS + 2 * CH + g * L, L)].astype(jnp.float32)
            for ii in range(L):
                svi = _bcast_lane(sv16, ii)
                row = g * L + ii
                for j in range(D // L):
                    sl = pl.ds(j * L, L)
                    rbufs[b, row, sl] = rbufs[b, row, sl] * svi
            return 0
        lax.fori_loop(0, CH // L, scale_group, 0)

        for j in range(CH // L):
            sl = pl.ds(j * L, L)
            dring[b, sl] = mring[pl.ds(b * MWORDS + CH + j * L, L)]
        pltpu.async_copy(rbufs.at[b], acc.at[dring.at[b]],
                         ssems.at[b], add=True)

    def outer(st, _):
        for b in range(_NBUF):
            iter_body(st * _NBUF + b, b)
        return 0
    lax.fori_loop(0, (NCH - 1) // _NBUF, outer, 0)
    iter_body(jnp.int32(NCH - 1), (NCH - 1) % _NBUF)

    s_wait((NCH - 2) % _NBUF)
    s_wait((NCH - 1) % _NBUF)

    plsc.subcore_barrier()

    for q in range(npt // _ZROWS):
        r0 = sid * npt + q * _ZROWS
        pltpu.async_copy(acc.at[pl.ds(r0, _ZROWS)],
                         p_hbm.at[cid, pl.ds(r0, _ZROWS)], wsem)
    for q in range(npt // _ZROWS):
        pltpu.make_async_copy(acc.at[pl.ds(sid * npt, _ZROWS)],
                              p_hbm.at[cid, pl.ds(sid * npt, _ZROWS)],
                              wsem).wait()


def _y_body(comp_ref, basis_ref, x_ref, y_ref):
    r = pl.program_id(0)
    w = comp_ref[r, 0] * basis_ref[0]
    for b in range(1, NB):
        w = w + comp_ref[r, b] * basis_ref[b]
    y_ref[...] = jnp.dot(x_ref[...], w, preferred_element_type=jnp.float32)


def _y_call(x, comp, basis):
    return pl.pallas_call(
        _y_body,
        grid=(R,),
        in_specs=[
            pl.BlockSpec(memory_space=pltpu.SMEM),
            pl.BlockSpec((NB, D, D), lambda r: (0, 0, 0)),
            pl.BlockSpec((N, D), lambda r: (0, 0)),
        ],
        out_specs=pl.BlockSpec((N, D), lambda r: (r, 0)),
        out_shape=jax.ShapeDtypeStruct((R * N, D), jnp.float32),
    )(comp, basis, x)


_BN = 1000

def _epi_body(relu, p_ref, x_ref, root_ref, bias_ref, o_ref):
    acc = (p_ref[0] + p_ref[1] + bias_ref[0]
           + jnp.dot(x_ref[...], root_ref[...], preferred_element_type=jnp.float32))
    if relu:
        acc = jnp.maximum(acc, 0.0)
    o_ref[...] = acc


def _epi_call(p, x, root, bias, relu):
    return pl.pallas_call(
        functools.partial(_epi_body, relu),
        grid=(N // _BN,),
        in_specs=[
            pl.BlockSpec((NC, _BN, D), lambda i: (0, i, 0)),
            pl.BlockSpec((_BN, D), lambda i: (i, 0)),
            pl.BlockSpec((D, D), lambda i: (0, 0)),
            pl.BlockSpec((1, D), lambda i: (0, 0)),
        ],
        out_specs=pl.BlockSpec((_BN, D), lambda i: (i, 0)),
        out_shape=jax.ShapeDtypeStruct((N, D), jnp.float32),
    )(p, x, root, bias.reshape(1, D))


def kernel(edge_index, edge_type, embedding, basis1, comp1, root1, bias1,
           basis2, comp2, root2, bias2):
    src = edge_index[0]
    dst = edge_index[1]
    y1 = _y_call(embedding, comp1, basis1)
    meta = _sc_prep(src, dst, edge_type)

    p1 = _sc_edge(y1, meta)
    x2 = _epi_call(p1, embedding, root1, bias1, relu=True)

    y2 = _y_call(x2, comp2, basis2)
    p2 = _sc_edge(y2, meta)
    return _epi_call(p2, x2, root2, bias2, relu=False)
